# baseline (speedup 1.0000x reference)
import functools

import jax
import jax.numpy as jnp
from jax import lax
from jax.experimental import pallas as pl
from jax.experimental.pallas import tpu as pltpu
from jax.experimental.pallas import tpu_sc as plsc

RADIUS = 0.15
R2 = RADIUS * RADIUS
MAXK = 16
NKP = 16
N, NS, DIN, DQ, DOUT = 2500, 10000, 128, 32, 128
NPAD = 2504
CPAD = 2560
BQ = 400
NSPAD = 10240
EP = NSPAD * MAXK
NW = 32
EPW = EP // NW
CH = 128
NCH = EPW // CH
BIG = 1e9


def _tables_body(x_ref, pos_ref, wpre_ref, bpre_ref, wshort_ref, bshort_ref,
                 kpt_ref, t1_ref, tp_ref, t3_ref):
    rows = lax.broadcasted_iota(jnp.int32, (NPAD, 1), 0)
    valid = rows < N
    xv = x_ref[...]
    t1 = jnp.dot(xv, wpre_ref[...], preferred_element_type=jnp.float32)
    t1 = t1 + bpre_ref[...]
    t3 = jnp.dot(xv, wshort_ref[...], preferred_element_type=jnp.float32)
    t3 = t3 + bshort_ref[...]
    tp = (pos_ref[:, 0:1] * kpt_ref[0:1, :]
          + pos_ref[:, 1:2] * kpt_ref[1:2, :]
          + pos_ref[:, 2:3] * kpt_ref[2:3, :])
    t1_ref[...] = jnp.where(valid, t1, 0.0)
    tp_ref[...] = jnp.where(valid, tp, 0.0)
    t3_ref[...] = jnp.where(valid, t3, 0.0)


def _make_tables(x_pad, pos_pad, w_pre, b_pre, w_short, b_short, kpt):
    return pl.pallas_call(
        _tables_body,
        out_shape=(
            jax.ShapeDtypeStruct((NPAD, DQ), jnp.float32),
            jax.ShapeDtypeStruct((NPAD, NKP), jnp.float32),
            jax.ShapeDtypeStruct((NPAD, DOUT), jnp.float32),
        ),
    )(x_pad, pos_pad, w_pre, b_pre, w_short, b_short, kpt)


def _topk_body(ps_ref, post_ref, cols_ref, d2s_ref):
    ps = ps_ref[...]
    p0 = post_ref[0:1, :]
    p1 = post_ref[1:2, :]
    p2 = post_ref[2:3, :]
    d2 = ((ps[:, 0:1] - p0) ** 2 + (ps[:, 1:2] - p1) ** 2
          + (ps[:, 2:3] - p2) ** 2)
    d2m = jnp.where(d2 <= R2, d2, BIG)
    iota = lax.broadcasted_iota(jnp.int32, (BQ, CPAD), 1)
    cols_l = []
    d2s_l = []
    m_prev = jnp.full((BQ, 1), -1.0, jnp.float32)
    am_prev = jnp.full((BQ, 1), -1, jnp.int32)
    for _ in range(MAXK):
        live = (d2m > m_prev) | ((d2m == m_prev) & (iota > am_prev))
        m = jnp.min(jnp.where(live, d2m, BIG), axis=1, keepdims=True)
        am = jnp.min(jnp.where(live & (d2m == m), iota, CPAD), axis=1,
                     keepdims=True)
        cols_l.append(jnp.where(m < BIG, am, N).astype(jnp.int32))
        d2s_l.append(m)
        m_prev, am_prev = m, am
    cols_ref[...] = jnp.concatenate(cols_l, axis=1)
    d2s_ref[...] = jnp.concatenate(d2s_l, axis=1)


def _topk(pos_skip, post):
    return pl.pallas_call(
        _topk_body,
        grid=(NS // BQ,),
        in_specs=[
            pl.BlockSpec((BQ, 3), lambda i: (i, 0)),
            pl.BlockSpec((8, CPAD), lambda i: (0, 0)),
        ],
        out_specs=(
            pl.BlockSpec((BQ, MAXK), lambda i: (i, 0)),
            pl.BlockSpec((BQ, MAXK), lambda i: (i, 0)),
        ),
        out_shape=(
            jax.ShapeDtypeStruct((NS, MAXK), jnp.int32),
            jax.ShapeDtypeStruct((NS, MAXK), jnp.float32),
        ),
    )(pos_skip, post)


_sc_mesh = plsc.VectorSubcoreMesh(core_axis_name="c", subcore_axis_name="s")


@functools.partial(
    pl.kernel,
    mesh=_sc_mesh,
    out_type=(
        jax.ShapeDtypeStruct((EP, DQ), jnp.float32),
        jax.ShapeDtypeStruct((EP, NKP), jnp.float32),
        jax.ShapeDtypeStruct((EP, DOUT), jnp.float32),
    ),
    scratch_types=[
        pltpu.VMEM((EPW,), jnp.int32),
        [pltpu.VMEM((CH, DQ), jnp.float32) for _ in range(4)],
        [pltpu.VMEM((CH, NKP), jnp.float32) for _ in range(4)],
        [pltpu.VMEM((CH, DOUT), jnp.float32) for _ in range(4)],
        [pltpu.SemaphoreType.DMA for _ in range(4)],
    ],
    compiler_params=pltpu.CompilerParams(use_tc_tiling_on_sc=False),
)
def _sc_gather(t1_hbm, tp_hbm, t3_hbm, idx_hbm, o1_hbm, op_hbm, o3_hbm,
               idx_v, b1s, bps, b3s, sems):
    wid = lax.axis_index("s") * 2 + lax.axis_index("c")
    base = wid * EPW
    pltpu.sync_copy(idx_hbm.at[pl.ds(base, EPW)], idx_v)

    def start(slot, local_off):
        ix = idx_v.at[pl.ds(local_off, CH)]
        pltpu.async_copy(t1_hbm.at[ix], b1s[slot], sems[slot])
        pltpu.async_copy(tp_hbm.at[ix], bps[slot], sems[slot])
        pltpu.async_copy(t3_hbm.at[ix], b3s[slot], sems[slot])

    def wait(slot, local_off):
        ix = idx_v.at[pl.ds(local_off, CH)]
        pltpu.make_async_copy(t1_hbm.at[ix], b1s[slot], sems[slot]).wait()
        pltpu.make_async_copy(tp_hbm.at[ix], bps[slot], sems[slot]).wait()
        pltpu.make_async_copy(t3_hbm.at[ix], b3s[slot], sems[slot]).wait()

    for b in range(4):
        start(b, b * CH)

    def body(i, carry):
        for b in range(4):
            c = i * 4 + b
            off = c * CH
            wait(b, off)
            pltpu.sync_copy(b1s[b], o1_hbm.at[pl.ds(base + off, CH)])
            pltpu.sync_copy(bps[b], op_hbm.at[pl.ds(base + off, CH)])
            pltpu.sync_copy(b3s[b], o3_hbm.at[pl.ds(base + off, CH)])

            @pl.when(c + 4 < NCH)
            def _():
                start(b, off + 4 * CH)
        return carry

    lax.fori_loop(0, NCH // 4, body, 0)


def _fuse_body(g32_ref, gp_ref, gxs_ref, d2s_ref, ps_ref, xsk_ref, kpt_ref,
               kwf_ref, wpost_ref, bpost_ref, wmlp_ref, bmlp_ref, o_ref):
    ps = ps_ref[...]
    pssum = (ps[:, 0:1] * kpt_ref[0:1, :] + ps[:, 1:2] * kpt_ref[1:2, :]
             + ps[:, 2:3] * kpt_ref[2:3, :])
    kp2 = (kpt_ref[0:1, :] ** 2 + kpt_ref[1:2, :] ** 2
           + kpt_ref[2:3, :] ** 2)
    cterm = 2.0 * pssum + kp2
    agg = [jnp.zeros((BQ, DQ), jnp.float32) for _ in range(NKP)]
    shortcut = jnp.zeros((BQ, DOUT), jnp.float32)
    for e in range(MAXK):
        sqd = d2s_ref[:, e:e + 1] - 2.0 * gp_ref[:, e, :] + cterm
        w_e = jnp.maximum(1.0 - jnp.sqrt(jnp.maximum(sqd, 1e-12)) / RADIUS,
                          0.0)
        g32_e = g32_ref[:, e, :]
        for k in range(NKP):
            agg[k] = agg[k] + w_e[:, k:k + 1] * g32_e
        shortcut = shortcut + gxs_ref[:, e, :]
    aggf = jnp.concatenate(agg, axis=1)
    conv = jnp.dot(aggf, kwf_ref[...], preferred_element_type=jnp.float32)
    side = jnp.dot(conv, wpost_ref[...], preferred_element_type=jnp.float32)
    side = side + bpost_ref[...]
    o1 = side + shortcut
    y = (jnp.dot(o1, wmlp_ref[0:DOUT, :], preferred_element_type=jnp.float32)
         + jnp.dot(xsk_ref[...], wmlp_ref[DOUT:DOUT + DIN, :],
                   preferred_element_type=jnp.float32)
         + bmlp_ref[...])
    o_ref[...] = jnp.where(y >= 0.0, y, 0.2 * y)


def _fuse(g32r, gpr, gxsr, d2s, pos_skip, x_skip, kpt, kwf, w_post, b_post,
          w_mlp, b_mlp):
    return pl.pallas_call(
        _fuse_body,
        grid=(NS // BQ,),
        in_specs=[
            pl.BlockSpec((BQ, MAXK, DQ), lambda i: (i, 0, 0)),
            pl.BlockSpec((BQ, MAXK, NKP), lambda i: (i, 0, 0)),
            pl.BlockSpec((BQ, MAXK, DOUT), lambda i: (i, 0, 0)),
            pl.BlockSpec((BQ, MAXK), lambda i: (i, 0)),
            pl.BlockSpec((BQ, 3), lambda i: (i, 0)),
            pl.BlockSpec((BQ, DIN), lambda i: (i, 0)),
            pl.BlockSpec((8, NKP), lambda i: (0, 0)),
            pl.BlockSpec((NKP * DQ, DQ), lambda i: (0, 0)),
            pl.BlockSpec((DQ, DOUT), lambda i: (0, 0)),
            pl.BlockSpec((1, DOUT), lambda i: (0, 0)),
            pl.BlockSpec((DOUT + DIN, DOUT), lambda i: (0, 0)),
            pl.BlockSpec((1, DOUT), lambda i: (0, 0)),
        ],
        out_specs=pl.BlockSpec((BQ, DOUT), lambda i: (i, 0)),
        out_shape=jax.ShapeDtypeStruct((NS, DOUT), jnp.float32),
    )(g32r, gpr, gxsr, d2s, pos_skip, x_skip, kpt, kwf, w_post, b_post,
      w_mlp, b_mlp)


def kernel(x, pos, batch, x_skip, pos_skip, batch_skip, W_pre, b_pre,
           kernel_pts, kernel_weight, W_post, b_post, W_short, b_short,
           W_mlp, b_mlp):
    f32 = jnp.float32
    x_pad = jnp.pad(x, ((0, NPAD - N), (0, 0)))
    pos_pad = jnp.pad(pos, ((0, NPAD - N), (0, 0)))
    kpt = jnp.zeros((8, NKP), f32).at[:3, :].set(kernel_pts.T)
    post = jnp.full((8, CPAD), 1e4, f32).at[:3, :N].set(pos.T)

    t1, tp, t3 = _make_tables(x_pad, pos_pad, W_pre,
                              b_pre.reshape(1, DQ).astype(f32), W_short,
                              b_short.reshape(1, DOUT).astype(f32), kpt)
    cols, d2s = _topk(pos_skip, post)

    cols_pad = jnp.pad(cols, ((0, NSPAD - NS), (0, 0)), constant_values=N)
    g32, gp, gxs = _sc_gather(t1, tp, t3, cols_pad.reshape(EP))

    out = _fuse(g32.reshape(NSPAD, MAXK, DQ),
                gp.reshape(NSPAD, MAXK, NKP),
                gxs.reshape(NSPAD, MAXK, DOUT),
                d2s, pos_skip, x_skip, kpt,
                kernel_weight.reshape(NKP * DQ, DQ), W_post,
                b_post.reshape(1, DOUT).astype(f32), W_mlp,
                b_mlp.reshape(1, DOUT).astype(f32))
    return out


# speedup vs baseline: 1.1989x; 1.1989x over previous
import functools

import jax
import jax.numpy as jnp
from jax import lax
from jax.experimental import pallas as pl
from jax.experimental.pallas import tpu as pltpu
from jax.experimental.pallas import tpu_sc as plsc

RADIUS = 0.15
R2 = RADIUS * RADIUS
MAXK = 16
NKP = 16
N, NS, DIN, DQ, DOUT = 2500, 10000, 128, 32, 128
NPAD = 2504
CPAD = 2560
BQ = 400
NSPAD = 10240
EP = NSPAD * MAXK
NW = 32
EPW = EP // NW
CH = 128
NCH = EPW // CH
BIG = 1e9


def _tables_body(x_ref, pos_ref, wpre_ref, bpre_ref, wshort_ref, bshort_ref,
                 kpt_ref, t1_ref, tp_ref, t3_ref):
    rows = lax.broadcasted_iota(jnp.int32, (NPAD, 1), 0)
    valid = rows < N
    xv = x_ref[...]
    t1 = jnp.dot(xv, wpre_ref[...], preferred_element_type=jnp.float32)
    t1 = t1 + bpre_ref[...]
    t3 = jnp.dot(xv, wshort_ref[...], preferred_element_type=jnp.float32)
    t3 = t3 + bshort_ref[...]
    tp = (pos_ref[:, 0:1] * kpt_ref[0:1, :]
          + pos_ref[:, 1:2] * kpt_ref[1:2, :]
          + pos_ref[:, 2:3] * kpt_ref[2:3, :])
    t1_ref[...] = jnp.where(valid, t1, 0.0)
    tp_ref[...] = jnp.where(valid, tp, 0.0)
    t3_ref[...] = jnp.where(valid, t3, 0.0)


def _make_tables(x_pad, pos_pad, w_pre, b_pre, w_short, b_short, kpt):
    return pl.pallas_call(
        _tables_body,
        out_shape=(
            jax.ShapeDtypeStruct((NPAD, DQ), jnp.float32),
            jax.ShapeDtypeStruct((NPAD, NKP), jnp.float32),
            jax.ShapeDtypeStruct((NPAD, DOUT), jnp.float32),
        ),
    )(x_pad, pos_pad, w_pre, b_pre, w_short, b_short, kpt)


def _topk_body(ps_ref, post_ref, cols_ref, d2s_ref):
    ps = ps_ref[...]
    p0 = post_ref[0:1, :]
    p1 = post_ref[1:2, :]
    p2 = post_ref[2:3, :]
    d2 = ((ps[:, 0:1] - p0) ** 2 + (ps[:, 1:2] - p1) ** 2
          + (ps[:, 2:3] - p2) ** 2)
    d2m = jnp.where(d2 <= R2, d2, BIG)
    iota = lax.broadcasted_iota(jnp.int32, (BQ, CPAD), 1)
    cols_l = []
    d2s_l = []
    for _ in range(MAXK):
        m = jnp.min(d2m, axis=1, keepdims=True)
        am = jnp.min(jnp.where(d2m == m, iota, CPAD), axis=1, keepdims=True)
        cols_l.append(jnp.where(m < BIG, am, N).astype(jnp.int32))
        d2s_l.append(m)
        d2m = jnp.where(iota == am, BIG, d2m)
    cols_ref[...] = jnp.concatenate(cols_l, axis=1)
    d2s_ref[...] = jnp.concatenate(d2s_l, axis=1)


def _topk(pos_skip, post):
    return pl.pallas_call(
        _topk_body,
        grid=(NS // BQ,),
        in_specs=[
            pl.BlockSpec((BQ, 3), lambda i: (i, 0)),
            pl.BlockSpec((8, CPAD), lambda i: (0, 0)),
        ],
        out_specs=(
            pl.BlockSpec((BQ, MAXK), lambda i: (i, 0)),
            pl.BlockSpec((BQ, MAXK), lambda i: (i, 0)),
        ),
        out_shape=(
            jax.ShapeDtypeStruct((NS, MAXK), jnp.int32),
            jax.ShapeDtypeStruct((NS, MAXK), jnp.float32),
        ),
    )(pos_skip, post)


_sc_mesh = plsc.VectorSubcoreMesh(core_axis_name="c", subcore_axis_name="s")


@functools.partial(
    pl.kernel,
    mesh=_sc_mesh,
    out_type=(
        jax.ShapeDtypeStruct((EP, DQ), jnp.float32),
        jax.ShapeDtypeStruct((EP, NKP), jnp.float32),
        jax.ShapeDtypeStruct((EP, DOUT), jnp.float32),
    ),
    scratch_types=[
        pltpu.VMEM((EPW,), jnp.int32),
        [pltpu.VMEM((CH, DQ), jnp.float32) for _ in range(4)],
        [pltpu.VMEM((CH, NKP), jnp.float32) for _ in range(4)],
        [pltpu.VMEM((CH, DOUT), jnp.float32) for _ in range(4)],
        [pltpu.SemaphoreType.DMA for _ in range(4)],
    ],
    compiler_params=pltpu.CompilerParams(use_tc_tiling_on_sc=False),
)
def _sc_gather(t1_hbm, tp_hbm, t3_hbm, idx_hbm, o1_hbm, op_hbm, o3_hbm,
               idx_v, b1s, bps, b3s, sems):
    wid = lax.axis_index("s") * 2 + lax.axis_index("c")
    base = wid * EPW
    pltpu.sync_copy(idx_hbm.at[pl.ds(base, EPW)], idx_v)

    def start(slot, local_off):
        ix = idx_v.at[pl.ds(local_off, CH)]
        pltpu.async_copy(t1_hbm.at[ix], b1s[slot], sems[slot])
        pltpu.async_copy(tp_hbm.at[ix], bps[slot], sems[slot])
        pltpu.async_copy(t3_hbm.at[ix], b3s[slot], sems[slot])

    def wait(slot, local_off):
        ix = idx_v.at[pl.ds(local_off, CH)]
        pltpu.make_async_copy(t1_hbm.at[ix], b1s[slot], sems[slot]).wait()
        pltpu.make_async_copy(tp_hbm.at[ix], bps[slot], sems[slot]).wait()
        pltpu.make_async_copy(t3_hbm.at[ix], b3s[slot], sems[slot]).wait()

    for b in range(4):
        start(b, b * CH)

    def body(i, carry):
        for b in range(4):
            c = i * 4 + b
            off = c * CH
            wait(b, off)
            pltpu.sync_copy(b1s[b], o1_hbm.at[pl.ds(base + off, CH)])
            pltpu.sync_copy(bps[b], op_hbm.at[pl.ds(base + off, CH)])
            pltpu.sync_copy(b3s[b], o3_hbm.at[pl.ds(base + off, CH)])

            @pl.when(c + 4 < NCH)
            def _():
                start(b, off + 4 * CH)
        return carry

    lax.fori_loop(0, NCH // 4, body, 0)


def _fuse_body(g32_ref, gp_ref, gxs_ref, d2s_ref, ps_ref, xsk_ref, kpt_ref,
               kwf_ref, wpost_ref, bpost_ref, wmlp_ref, bmlp_ref, o_ref):
    ps = ps_ref[...]
    pssum = (ps[:, 0:1] * kpt_ref[0:1, :] + ps[:, 1:2] * kpt_ref[1:2, :]
             + ps[:, 2:3] * kpt_ref[2:3, :])
    kp2 = (kpt_ref[0:1, :] ** 2 + kpt_ref[1:2, :] ** 2
           + kpt_ref[2:3, :] ** 2)
    cterm = 2.0 * pssum + kp2
    agg = [jnp.zeros((BQ, DQ), jnp.float32) for _ in range(NKP)]
    shortcut = jnp.zeros((BQ, DOUT), jnp.float32)
    for e in range(MAXK):
        sqd = d2s_ref[:, e:e + 1] - 2.0 * gp_ref[:, e, :] + cterm
        w_e = jnp.maximum(1.0 - jnp.sqrt(jnp.maximum(sqd, 1e-12)) / RADIUS,
                          0.0)
        g32_e = g32_ref[:, e, :]
        for k in range(NKP):
            agg[k] = agg[k] + w_e[:, k:k + 1] * g32_e
        shortcut = shortcut + gxs_ref[:, e, :]
    aggf = jnp.concatenate(agg, axis=1)
    conv = jnp.dot(aggf, kwf_ref[...], preferred_element_type=jnp.float32)
    side = jnp.dot(conv, wpost_ref[...], preferred_element_type=jnp.float32)
    side = side + bpost_ref[...]
    o1 = side + shortcut
    y = (jnp.dot(o1, wmlp_ref[0:DOUT, :], preferred_element_type=jnp.float32)
         + jnp.dot(xsk_ref[...], wmlp_ref[DOUT:DOUT + DIN, :],
                   preferred_element_type=jnp.float32)
         + bmlp_ref[...])
    o_ref[...] = jnp.where(y >= 0.0, y, 0.2 * y)


def _fuse(g32r, gpr, gxsr, d2s, pos_skip, x_skip, kpt, kwf, w_post, b_post,
          w_mlp, b_mlp):
    return pl.pallas_call(
        _fuse_body,
        grid=(NS // BQ,),
        in_specs=[
            pl.BlockSpec((BQ, MAXK, DQ), lambda i: (i, 0, 0)),
            pl.BlockSpec((BQ, MAXK, NKP), lambda i: (i, 0, 0)),
            pl.BlockSpec((BQ, MAXK, DOUT), lambda i: (i, 0, 0)),
            pl.BlockSpec((BQ, MAXK), lambda i: (i, 0)),
            pl.BlockSpec((BQ, 3), lambda i: (i, 0)),
            pl.BlockSpec((BQ, DIN), lambda i: (i, 0)),
            pl.BlockSpec((8, NKP), lambda i: (0, 0)),
            pl.BlockSpec((NKP * DQ, DQ), lambda i: (0, 0)),
            pl.BlockSpec((DQ, DOUT), lambda i: (0, 0)),
            pl.BlockSpec((1, DOUT), lambda i: (0, 0)),
            pl.BlockSpec((DOUT + DIN, DOUT), lambda i: (0, 0)),
            pl.BlockSpec((1, DOUT), lambda i: (0, 0)),
        ],
        out_specs=pl.BlockSpec((BQ, DOUT), lambda i: (i, 0)),
        out_shape=jax.ShapeDtypeStruct((NS, DOUT), jnp.float32),
    )(g32r, gpr, gxsr, d2s, pos_skip, x_skip, kpt, kwf, w_post, b_post,
      w_mlp, b_mlp)


def kernel(x, pos, batch, x_skip, pos_skip, batch_skip, W_pre, b_pre,
           kernel_pts, kernel_weight, W_post, b_post, W_short, b_short,
           W_mlp, b_mlp):
    f32 = jnp.float32
    x_pad = jnp.pad(x, ((0, NPAD - N), (0, 0)))
    pos_pad = jnp.pad(pos, ((0, NPAD - N), (0, 0)))
    kpt = jnp.zeros((8, NKP), f32).at[:3, :].set(kernel_pts.T)
    post = jnp.full((8, CPAD), 1e4, f32).at[:3, :N].set(pos.T)

    t1, tp, t3 = _make_tables(x_pad, pos_pad, W_pre,
                              b_pre.reshape(1, DQ).astype(f32), W_short,
                              b_short.reshape(1, DOUT).astype(f32), kpt)
    cols, d2s = _topk(pos_skip, post)

    cols_pad = jnp.pad(cols, ((0, NSPAD - NS), (0, 0)), constant_values=N)
    g32, gp, gxs = _sc_gather(t1, tp, t3, cols_pad.reshape(EP))

    out = _fuse(g32.reshape(NSPAD, MAXK, DQ),
                gp.reshape(NSPAD, MAXK, NKP),
                gxs.reshape(NSPAD, MAXK, DOUT),
                d2s, pos_skip, x_skip, kpt,
                kernel_weight.reshape(NKP * DQ, DQ), W_post,
                b_post.reshape(1, DOUT).astype(f32), W_mlp,
                b_mlp.reshape(1, DOUT).astype(f32))
    return out
